# Initial kernel scaffold; baseline (speedup 1.0000x reference)
import jax
import jax.numpy as jnp
from jax.experimental import pallas as pl


def kernel(x, params, edge_index):
    raise NotImplementedError("write your pallas kernel here")


# speedup vs baseline: 1.0069x; 1.0069x over previous
import jax
import jax.numpy as jnp
from jax.experimental import pallas as pl

_HID = 64
_R = 2000


def _elu(v):
    return jnp.where(v > 0, v, jnp.exp(v) - 1.0)


def _rows(c):
    return pl.BlockSpec((_R, c), lambda i: (i, 0))


def _full(r, c):
    return pl.BlockSpec((r, c), lambda i: (0, 0))


def _ka(x_ref, w_ref, b_ref, o_ref):
    o_ref[...] = (
        jnp.dot(x_ref[...], w_ref[...], preferred_element_type=jnp.float32)
        + b_ref[...]
    )


def _kb(hh_ref, pr_ref, sk_ref, v_ref, w_ref, o1_ref, o2_ref):
    hh = hh_ref[...]
    hmean = 0.25 * (hh[:, :64] + hh[:, 64:128] + hh[:, 128:192] + hh[:, 192:256])
    hmean = hmean + v_ref[0:1]
    hm = hmean * v_ref[1:2] + pr_ref[...] * v_ref[2:3]
    h1 = _elu(hm * v_ref[3:4] + v_ref[4:5]) + sk_ref[...]
    o1_ref[...] = h1
    o2_ref[...] = jnp.dot(h1, w_ref[...], preferred_element_type=jnp.float32)


def _kc(agg_ref, h_ref, v_ref, w_ref, o1_ref, o2_ref):
    h2 = _elu((agg_ref[...] + v_ref[0:1]) * v_ref[1:2] + v_ref[2:3])
    h2 = h2 + h_ref[...]
    o1_ref[...] = h2
    o2_ref[...] = jnp.dot(h2, w_ref[...], preferred_element_type=jnp.float32)


def _kd(hh_ref, h_ref, v_ref, w_ref, o1_ref, o2_ref):
    hh = hh_ref[...]
    hmean = 0.5 * (hh[:, :64] + hh[:, 64:128]) + v_ref[0:1]
    h3 = _elu(hmean * v_ref[1:2] + v_ref[2:3]) + h_ref[...]
    o1_ref[...] = h3
    o2_ref[...] = jnp.dot(h3, w_ref[...], preferred_element_type=jnp.float32)


def _ke(agg_ref, h_ref, nv_ref, v_ref, w1_ref, w2_ref, b2_ref, o_ref):
    h4 = _elu((agg_ref[...] + v_ref[0:1]) * v_ref[1:2] + v_ref[2:3])
    h4 = h4 + h_ref[...]
    t = jax.nn.relu(
        jnp.dot(h4, w1_ref[...], preferred_element_type=jnp.float32)
        + v_ref[3:4, :32]
    )
    raw = jnp.dot(t, w2_ref[...], preferred_element_type=jnp.float32) + b2_ref[...]
    o_ref[...] = jax.nn.sigmoid(raw + nv_ref[...][:, 8:9])


def _padrows(a, m):
    p = (-a.shape[0]) % m
    return jnp.pad(a, ((0, p), (0, 0))) if p else a


def _seg_softmax(logits, seg, n):
    m = jax.ops.segment_max(logits, seg, num_segments=n)
    m = jnp.where(jnp.isfinite(m), m, 0.0)
    e = jnp.exp(logits - m[seg])
    s = jax.ops.segment_sum(e, seg, num_segments=n)
    return e / (s[seg] + 1e-16)


def kernel(x, params, edge_index):
    p = params
    n = x.shape[0]
    f32 = jnp.float32
    c = (1.0 + 1e-5) ** 0.5
    g = jax.nn.sigmoid(p["gate"])

    np_ = n + ((-n) % _R)
    grid = (np_ // _R,)

    as_c = jnp.concatenate(
        [p["gat1_W"][:, h * 64 : (h + 1) * 64] @ p["gat1_as"][h][:, None] for h in range(4)],
        axis=1,
    )
    ad_c = jnp.concatenate(
        [p["gat1_W"][:, h * 64 : (h + 1) * 64] @ p["gat1_ad"][h][:, None] for h in range(4)],
        axis=1,
    )
    wbig = jnp.zeros((48, 512), f32)
    wbig = wbig.at[:43, :256].set(p["gat1_W"])
    wbig = wbig.at[:43, 256:320].set(p["skip_W"])
    wbig = wbig.at[11:43, 320:384].set(p["kg_prior_W"])
    wbig = wbig.at[:43, 384:388].set(as_c)
    wbig = wbig.at[:43, 388:392].set(ad_c)
    wbig = wbig.at[11:43, 392:393].set(p["kg_vuln"] * jax.nn.sigmoid(p["vuln_scale"]))
    bbig = jnp.zeros((1, 512), f32)
    bbig = bbig.at[0, 256:320].set(p["skip_b"])
    bbig = bbig.at[0, 320:384].set(p["kg_prior_b"])

    xp = _padrows(jnp.pad(x, ((0, 0), (0, 48 - x.shape[1]))), _R)
    y = pl.pallas_call(
        _ka,
        grid=grid,
        in_specs=[_rows(48), _full(48, 512), _full(1, 512)],
        out_specs=_rows(512),
        out_shape=jax.ShapeDtypeStruct((np_, 512), f32),
    )(xp, wbig, bbig)

    xl = y[:n, :256]
    skip = y[:n, 256:320]
    prior = y[:n, 320:384]
    as_ = y[:n, 384:388]
    ad_ = y[:n, 388:392]
    nvp = y[:, 384:512]

    src0, dst0 = edge_index[0], edge_index[1]
    loop = jnp.arange(n, dtype=src0.dtype)
    src = jnp.concatenate([src0, loop])
    dst = jnp.concatenate([dst0, loop])
    kg = x[:, -32:]
    kg_class = jnp.argmax(kg, axis=-1)
    same = (kg_class[src0] == kg_class[dst0]).astype(f32)
    eb = same * p["same_bias"]
    ea = jnp.concatenate([eb, jnp.full((n,), eb.mean(), dtype=f32)])
    cvec = jnp.sum(p["gat1_lin_edge"].reshape(4, 64) * p["gat1_ae"], axis=-1)
    aev = ea[:, None] * cvec[None, :]

    alpha = as_[src] + ad_[dst] + aev
    alpha = jax.nn.leaky_relu(alpha, 0.2)
    alpha = _seg_softmax(alpha, dst, n)
    xl_r = xl.reshape(n, 4, 64)
    heads = jax.ops.segment_sum(xl_r[src] * alpha[:, :, None], dst, num_segments=n)
    hh = _padrows(heads.reshape(n, 256), _R)

    vb = jnp.stack(
        [
            jnp.broadcast_to(p["gat1_b"], (64,)),
            jnp.full((64,), 1.0 - g, f32),
            jnp.full((64,), g, f32),
            p["bn1_g"] / c,
            p["bn1_b"],
            jnp.zeros((64,), f32),
            jnp.zeros((64,), f32),
            jnp.zeros((64,), f32),
        ]
    )
    h1, xw2 = pl.pallas_call(
        _kb,
        grid=grid,
        in_specs=[_rows(256), _rows(64), _rows(64), _full(8, 64), _full(64, 64)],
        out_specs=(_rows(64), _rows(64)),
        out_shape=(
            jax.ShapeDtypeStruct((np_, 64), f32),
            jax.ShapeDtypeStruct((np_, 64), f32),
        ),
    )(hh, _padrows(prior, _R), _padrows(skip, _R), vb, p["gcn2_W"])
    h1 = h1[:n]

    deg = jax.ops.segment_sum(jnp.ones_like(dst, dtype=f32), dst, num_segments=n)
    dis = jnp.where(deg > 0, deg**-0.5, 0.0)
    norm = dis[src] * dis[dst]
    agg2 = jax.ops.segment_sum(norm[:, None] * xw2[:n][src], dst, num_segments=n)

    as3_c = jnp.concatenate(
        [p["gat3_W"][:, h * 64 : (h + 1) * 64] @ p["gat3_as"][h][:, None] for h in range(2)],
        axis=1,
    )
    ad3_c = jnp.concatenate(
        [p["gat3_W"][:, h * 64 : (h + 1) * 64] @ p["gat3_ad"][h][:, None] for h in range(2)],
        axis=1,
    )
    w3 = jnp.zeros((64, 256), f32)
    w3 = w3.at[:, :128].set(p["gat3_W"])
    w3 = w3.at[:, 128:130].set(as3_c)
    w3 = w3.at[:, 130:132].set(ad3_c)
    vc = jnp.stack([p["gcn2_b"], p["bn2_g"] / c, p["bn2_b"]] + [jnp.zeros((64,), f32)] * 5)
    h2, y3 = pl.pallas_call(
        _kc,
        grid=grid,
        in_specs=[_rows(64), _rows(64), _full(8, 64), _full(64, 256)],
        out_specs=(_rows(64), _rows(256)),
        out_shape=(
            jax.ShapeDtypeStruct((np_, 64), f32),
            jax.ShapeDtypeStruct((np_, 256), f32),
        ),
    )(_padrows(agg2, _R), _padrows(h1, _R), vc, w3)
    h2 = h2[:n]
    xl3 = y3[:n, :128]
    as3 = y3[:n, 128:130]
    ad3 = y3[:n, 130:132]

    alpha3 = jax.nn.leaky_relu(as3[src] + ad3[dst], 0.2)
    alpha3 = _seg_softmax(alpha3, dst, n)
    heads3 = jax.ops.segment_sum(
        xl3.reshape(n, 2, 64)[src] * alpha3[:, :, None], dst, num_segments=n
    )
    hh3 = _padrows(heads3.reshape(n, 128), _R)

    vd = jnp.stack([p["gat3_b"], p["bn3_g"] / c, p["bn3_b"]] + [jnp.zeros((64,), f32)] * 5)
    h3, xw4 = pl.pallas_call(
        _kd,
        grid=grid,
        in_specs=[_rows(128), _rows(64), _full(8, 64), _full(64, 64)],
        out_specs=(_rows(64), _rows(64)),
        out_shape=(
            jax.ShapeDtypeStruct((np_, 64), f32),
            jax.ShapeDtypeStruct((np_, 64), f32),
        ),
    )(hh3, _padrows(h2, _R), vd, p["gcn4_W"])
    h3 = h3[:n]
    agg4 = jax.ops.segment_sum(norm[:, None] * xw4[:n][src], dst, num_segments=n)

    b1p = jnp.zeros((64,), f32).at[:32].set(p["ro_b1"])
    ve = jnp.stack(
        [p["gcn4_b"], p["bn4_g"] / c, p["bn4_b"], b1p] + [jnp.zeros((64,), f32)] * 4
    )
    w2p = jnp.zeros((32, 128), f32).at[:, 0].set(p["ro_W2"][:, 0])
    b2p = jnp.zeros((1, 128), f32).at[0, 0].set(p["ro_b2"][0])
    ye = pl.pallas_call(
        _ke,
        grid=grid,
        in_specs=[
            _rows(64),
            _rows(64),
            _rows(128),
            _full(8, 64),
            _full(64, 32),
            _full(32, 128),
            _full(1, 128),
        ],
        out_specs=_rows(128),
        out_shape=jax.ShapeDtypeStruct((np_, 128), f32),
    )(_padrows(agg4, _R), _padrows(h3, _R), nvp, ve, p["ro_W1"], w2p, b2p)
    return ye[:n, 0]
